# baseline (speedup 1.0000x reference)
import functools

import jax
import jax.numpy as jnp
from jax import lax
from jax.experimental import pallas as pl
from jax.experimental.pallas import tpu as pltpu
from jax.experimental.pallas import tpu_sc as plsc

TOP_K = 2
NUM_EXPERTS = 64
HIDDEN = 2048
TOKENS = 8192

NC = 2
NS = 16
LANES = 16
NW = NC * NS
CHUNK = TOKENS // NW
GROUPS = CHUNK // LANES


def _tc_logits_block(x_ref, w_ref, b_ref, out_ref):
    logits_t = lax.dot_general(
        w_ref[...], x_ref[...], (((1,), (1,)), ((), ())),
        preferred_element_type=jnp.float32)
    out_ref[...] = (logits_t + b_ref[...]).reshape(1, NUM_EXPERTS, CHUNK)


def _tc_logits(x, W, b):
    return pl.pallas_call(
        _tc_logits_block,
        grid=(NW,),
        in_specs=[
            pl.BlockSpec((CHUNK, HIDDEN), lambda i: (i, 0)),
            pl.BlockSpec((NUM_EXPERTS, HIDDEN), lambda i: (0, 0)),
            pl.BlockSpec((NUM_EXPERTS, 1), lambda i: (0, 0)),
        ],
        out_specs=pl.BlockSpec((1, NUM_EXPERTS, CHUNK), lambda i: (i, 0, 0)),
        out_shape=jax.ShapeDtypeStruct((NW, NUM_EXPERTS, CHUNK), jnp.float32),
    )(x, W, b.reshape(NUM_EXPERTS, 1))


def _sc_router_body(logits_hbm, scores_hbm, idx_hbm, logits_v, scores_v, idx_v):
    wid = lax.axis_index("s") * NC + lax.axis_index("c")
    pltpu.sync_copy(logits_hbm.at[wid], logits_v)

    iota = lax.broadcasted_iota(jnp.int32, (LANES,), 0)
    zf = jnp.zeros((LANES,), jnp.float32)

    def group_body(g, carry):
        t0 = g * LANES
        m1 = logits_v[0, pl.ds(t0, LANES)]
        i1 = jnp.zeros((LANES,), jnp.int32)
        m2 = jnp.full((LANES,), -jnp.inf, jnp.float32)
        i2 = jnp.zeros((LANES,), jnp.int32)
        for e in range(1, NUM_EXPERTS):
            v = logits_v[e, pl.ds(t0, LANES)]
            ev = jnp.full((LANES,), e, jnp.int32)
            c1 = v > m1
            c2 = v > m2
            m2 = jnp.where(c1, m1, jnp.where(c2, v, m2))
            i2 = jnp.where(c1, i1, jnp.where(c2, ev, i2))
            m1 = jnp.where(c1, v, m1)
            i1 = jnp.where(c1, ev, i1)
        d = jnp.exp(m2 - m1)
        p1 = 1.0 / (1.0 + d)
        p2 = d * p1
        base = t0 * NUM_EXPERTS
        for j in range(NUM_EXPERTS):
            scores_v[pl.ds(base + j * LANES, LANES)] = zf
        rows = (t0 + iota) * NUM_EXPERTS
        plsc.store_scatter(scores_v, [rows + i1], p1)
        plsc.store_scatter(scores_v, [rows + i2], p2)
        tok2 = (t0 + iota) * TOP_K
        plsc.store_scatter(idx_v, [tok2], i1)
        plsc.store_scatter(idx_v, [tok2 + 1], i2)
        return carry

    lax.fori_loop(0, GROUPS, group_body, 0)

    pltpu.sync_copy(scores_v, scores_hbm.at[pl.ds(wid * CHUNK * NUM_EXPERTS,
                                                  CHUNK * NUM_EXPERTS)])
    pltpu.sync_copy(idx_v, idx_hbm.at[pl.ds(wid * CHUNK * TOP_K,
                                            CHUNK * TOP_K)])


_sc_router = functools.partial(
    pl.kernel,
    out_type=(
        jax.ShapeDtypeStruct((TOKENS * NUM_EXPERTS,), jnp.float32),
        jax.ShapeDtypeStruct((TOKENS * TOP_K,), jnp.int32),
    ),
    mesh=plsc.VectorSubcoreMesh(core_axis_name="c", subcore_axis_name="s",
                                num_cores=NC, num_subcores=NS),
    scratch_types=[
        pltpu.VMEM((NUM_EXPERTS, CHUNK), jnp.float32),
        pltpu.VMEM((CHUNK * NUM_EXPERTS,), jnp.float32),
        pltpu.VMEM((CHUNK * TOP_K,), jnp.int32),
    ],
    compiler_params=pltpu.CompilerParams(needs_layout_passes=False),
)(_sc_router_body)


def kernel(hidden_states, W, b):
    x = hidden_states.reshape(-1, HIDDEN)
    logits3 = _tc_logits(x, W, b)
    scores_flat, idx_flat = _sc_router(logits3)
    return (scores_flat.reshape(TOKENS, NUM_EXPERTS),
            idx_flat.reshape(TOKENS, TOP_K))


# speedup vs baseline: 1.0292x; 1.0292x over previous
import functools

import jax
import jax.numpy as jnp
from jax import lax
from jax.experimental import pallas as pl
from jax.experimental.pallas import tpu as pltpu
from jax.experimental.pallas import tpu_sc as plsc

TOP_K = 2
NUM_EXPERTS = 64
HIDDEN = 2048
TOKENS = 8192

NC = 2
NS = 16
LANES = 16
NW = NC * NS
CHUNK = TOKENS // NW
GROUPS = CHUNK // LANES


def _tc_topk_block(x_ref, w_ref, b_ref, p_ref, i_ref):
    logits = lax.dot_general(
        w_ref[...], x_ref[...], (((1,), (1,)), ((), ())),
        preferred_element_type=jnp.float32)
    logits = logits + b_ref[...]

    eidx = lax.broadcasted_iota(jnp.int32, logits.shape, 0)
    m1 = jnp.max(logits, axis=0, keepdims=True)
    i1 = jnp.min(jnp.where(logits == m1, eidx, NUM_EXPERTS), axis=0,
                 keepdims=True)
    masked = jnp.where(eidx == i1, -jnp.inf, logits)
    m2 = jnp.max(masked, axis=0, keepdims=True)
    i2 = jnp.min(jnp.where(masked == m2, eidx, NUM_EXPERTS), axis=0,
                 keepdims=True)

    d = jnp.exp(m2 - m1)
    p1 = 1.0 / (1.0 + d)
    p2 = d * p1

    p_ref[...] = jnp.concatenate([p1, p2], axis=0).reshape(1, TOP_K, CHUNK)
    i_ref[...] = jnp.concatenate([i1, i2], axis=0).reshape(1, TOP_K, CHUNK)


def _tc_topk(x, W, b):
    return pl.pallas_call(
        _tc_topk_block,
        grid=(NW,),
        in_specs=[
            pl.BlockSpec((CHUNK, HIDDEN), lambda i: (i, 0)),
            pl.BlockSpec((NUM_EXPERTS, HIDDEN), lambda i: (0, 0)),
            pl.BlockSpec((NUM_EXPERTS, 1), lambda i: (0, 0)),
        ],
        out_specs=[
            pl.BlockSpec((1, TOP_K, CHUNK), lambda i: (i, 0, 0)),
            pl.BlockSpec((1, TOP_K, CHUNK), lambda i: (i, 0, 0)),
        ],
        out_shape=[
            jax.ShapeDtypeStruct((NW, TOP_K, CHUNK), jnp.float32),
            jax.ShapeDtypeStruct((NW, TOP_K, CHUNK), jnp.int32),
        ],
    )(x, W, b.reshape(NUM_EXPERTS, 1))


def _sc_scatter_body(p_hbm, i_hbm, scores_hbm, idx_hbm, p_v, i_v, scores_v,
                     idx_v):
    wid = lax.axis_index("s") * NC + lax.axis_index("c")
    pltpu.sync_copy(p_hbm.at[wid], p_v)
    pltpu.sync_copy(i_hbm.at[wid], i_v)

    iota = lax.broadcasted_iota(jnp.int32, (LANES,), 0)
    zf = jnp.zeros((LANES,), jnp.float32)

    def group_body(g, carry):
        t0 = g * LANES
        base = t0 * NUM_EXPERTS
        for j in range(NUM_EXPERTS):
            scores_v[pl.ds(base + j * LANES, LANES)] = zf
        p1 = p_v[0, pl.ds(t0, LANES)]
        p2 = p_v[1, pl.ds(t0, LANES)]
        i1 = i_v[0, pl.ds(t0, LANES)]
        i2 = i_v[1, pl.ds(t0, LANES)]
        rows = (t0 + iota) * NUM_EXPERTS
        plsc.store_scatter(scores_v, [rows + i1], p1)
        plsc.store_scatter(scores_v, [rows + i2], p2)
        tok2 = (t0 + iota) * TOP_K
        plsc.store_scatter(idx_v, [tok2], i1)
        plsc.store_scatter(idx_v, [tok2 + 1], i2)
        return carry

    lax.fori_loop(0, GROUPS, group_body, 0)

    pltpu.sync_copy(scores_v, scores_hbm.at[pl.ds(wid * CHUNK * NUM_EXPERTS,
                                                  CHUNK * NUM_EXPERTS)])
    pltpu.sync_copy(idx_v, idx_hbm.at[pl.ds(wid * CHUNK * TOP_K,
                                            CHUNK * TOP_K)])


_sc_scatter = functools.partial(
    pl.kernel,
    out_type=(
        jax.ShapeDtypeStruct((TOKENS * NUM_EXPERTS,), jnp.float32),
        jax.ShapeDtypeStruct((TOKENS * TOP_K,), jnp.int32),
    ),
    mesh=plsc.VectorSubcoreMesh(core_axis_name="c", subcore_axis_name="s",
                                num_cores=NC, num_subcores=NS),
    scratch_types=[
        pltpu.VMEM((TOP_K, CHUNK), jnp.float32),
        pltpu.VMEM((TOP_K, CHUNK), jnp.int32),
        pltpu.VMEM((CHUNK * NUM_EXPERTS,), jnp.float32),
        pltpu.VMEM((CHUNK * TOP_K,), jnp.int32),
    ],
    compiler_params=pltpu.CompilerParams(needs_layout_passes=False),
)(_sc_scatter_body)


def kernel(hidden_states, W, b):
    x = hidden_states.reshape(-1, HIDDEN)
    pv, iv = _tc_topk(x, W, b)
    scores_flat, idx_flat = _sc_scatter(pv, iv)
    return (scores_flat.reshape(TOKENS, NUM_EXPERTS),
            idx_flat.reshape(TOKENS, TOP_K))


# speedup vs baseline: 1.0847x; 1.0539x over previous
import functools

import jax
import jax.numpy as jnp
from jax import lax
from jax.experimental import pallas as pl
from jax.experimental.pallas import tpu as pltpu
from jax.experimental.pallas import tpu_sc as plsc

TOP_K = 2
NUM_EXPERTS = 64
HIDDEN = 2048
TOKENS = 8192
BLOCK_T = 512

NC = 2
NS = 16
LANES = 16
NW = NC * NS
CHUNK = TOKENS // NW
GROUPS = CHUNK // LANES


def _tc_topk_block(x_ref, w_ref, b_ref, p_ref, i_ref):
    logits = lax.dot_general(
        x_ref[...], w_ref[...], (((1,), (1,)), ((), ())),
        preferred_element_type=jnp.float32)
    logits = logits + b_ref[...]

    eidx = lax.broadcasted_iota(jnp.int32, logits.shape, 1)
    m1 = jnp.max(logits, axis=1, keepdims=True)
    i1 = jnp.min(jnp.where(logits == m1, eidx, NUM_EXPERTS), axis=1,
                 keepdims=True)
    masked = jnp.where(eidx == i1, -jnp.inf, logits)
    m2 = jnp.max(masked, axis=1, keepdims=True)
    i2 = jnp.min(jnp.where(masked == m2, eidx, NUM_EXPERTS), axis=1,
                 keepdims=True)

    d = jnp.exp(m2 - m1)
    p1 = 1.0 / (1.0 + d)
    p2 = d * p1

    p_ref[...] = jnp.concatenate([p1, p2], axis=1)
    i_ref[...] = jnp.concatenate([i1, i2], axis=1)


def _tc_topk(x, W, b):
    return pl.pallas_call(
        _tc_topk_block,
        grid=(TOKENS // BLOCK_T,),
        in_specs=[
            pl.BlockSpec((BLOCK_T, HIDDEN), lambda i: (i, 0)),
            pl.BlockSpec((NUM_EXPERTS, HIDDEN), lambda i: (0, 0)),
            pl.BlockSpec((1, NUM_EXPERTS), lambda i: (0, 0)),
        ],
        out_specs=[
            pl.BlockSpec((BLOCK_T, TOP_K), lambda i: (i, 0)),
            pl.BlockSpec((BLOCK_T, TOP_K), lambda i: (i, 0)),
        ],
        out_shape=[
            jax.ShapeDtypeStruct((TOKENS, TOP_K), jnp.float32),
            jax.ShapeDtypeStruct((TOKENS, TOP_K), jnp.int32),
        ],
    )(x, W, b.reshape(1, NUM_EXPERTS))


def _sc_scatter_body(p_hbm, i_hbm, scores_hbm, p_v, i_v, scores_v):
    wid = lax.axis_index("s") * NC + lax.axis_index("c")
    pltpu.sync_copy(p_hbm.at[pl.ds(wid * CHUNK * TOP_K, CHUNK * TOP_K)], p_v)
    pltpu.sync_copy(i_hbm.at[pl.ds(wid * CHUNK * TOP_K, CHUNK * TOP_K)], i_v)

    iota = lax.broadcasted_iota(jnp.int32, (LANES,), 0)
    zf = jnp.zeros((LANES,), jnp.float32)

    def group_body(g, carry):
        t0 = g * LANES
        base = t0 * NUM_EXPERTS
        for j in range(NUM_EXPERTS):
            scores_v[pl.ds(base + j * LANES, LANES)] = zf
        pair = (t0 + iota) * TOP_K
        p1 = plsc.load_gather(p_v, [pair])
        p2 = plsc.load_gather(p_v, [pair + 1])
        i1 = plsc.load_gather(i_v, [pair])
        i2 = plsc.load_gather(i_v, [pair + 1])
        rows = (t0 + iota) * NUM_EXPERTS
        plsc.store_scatter(scores_v, [rows + i1], p1)
        plsc.store_scatter(scores_v, [rows + i2], p2)
        return carry

    lax.fori_loop(0, GROUPS, group_body, 0)

    pltpu.sync_copy(scores_v, scores_hbm.at[pl.ds(wid * CHUNK * NUM_EXPERTS,
                                                  CHUNK * NUM_EXPERTS)])


_sc_scatter = functools.partial(
    pl.kernel,
    out_type=jax.ShapeDtypeStruct((TOKENS * NUM_EXPERTS,), jnp.float32),
    mesh=plsc.VectorSubcoreMesh(core_axis_name="c", subcore_axis_name="s",
                                num_cores=NC, num_subcores=NS),
    scratch_types=[
        pltpu.VMEM((CHUNK * TOP_K,), jnp.float32),
        pltpu.VMEM((CHUNK * TOP_K,), jnp.int32),
        pltpu.VMEM((CHUNK * NUM_EXPERTS,), jnp.float32),
    ],
    compiler_params=pltpu.CompilerParams(needs_layout_passes=False),
)(_sc_scatter_body)


def kernel(hidden_states, W, b):
    x = hidden_states.reshape(-1, HIDDEN)
    pv, iv = _tc_topk(x, W, b)
    scores_flat = _sc_scatter(pv.reshape(-1), iv.reshape(-1))
    return scores_flat.reshape(TOKENS, NUM_EXPERTS), iv


# speedup vs baseline: 1.1615x; 1.0707x over previous
import functools

import jax
import jax.numpy as jnp
from jax import lax
from jax.experimental import pallas as pl
from jax.experimental.pallas import tpu as pltpu
from jax.experimental.pallas import tpu_sc as plsc

TOP_K = 2
NUM_EXPERTS = 64
HIDDEN = 2048
TOKENS = 8192
BLOCK_T = 512

NC = 2
NS = 16
LANES = 16
NW = NC * NS
CHUNK = TOKENS // NW
GROUPS = CHUNK // LANES


def _tc_topk_block(x_ref, w_ref, b_ref, p_ref, i_ref):
    logits = lax.dot_general(
        x_ref[...], w_ref[...], (((1,), (1,)), ((), ())),
        preferred_element_type=jnp.float32)
    logits = logits + b_ref[...]

    eidx = lax.broadcasted_iota(jnp.int32, logits.shape, 1)
    m1 = jnp.max(logits, axis=1, keepdims=True)
    i1 = jnp.min(jnp.where(logits == m1, eidx, NUM_EXPERTS), axis=1,
                 keepdims=True)
    masked = jnp.where(eidx == i1, -jnp.inf, logits)
    m2 = jnp.max(masked, axis=1, keepdims=True)
    i2 = jnp.min(jnp.where(masked == m2, eidx, NUM_EXPERTS), axis=1,
                 keepdims=True)

    d = jnp.exp(m2 - m1)
    p1 = 1.0 / (1.0 + d)
    p2 = d * p1

    p_ref[...] = jnp.concatenate([p1, p2], axis=1)
    i_ref[...] = jnp.concatenate([i1, i2], axis=1)


def _tc_topk(x, W, b):
    return pl.pallas_call(
        _tc_topk_block,
        grid=(TOKENS // BLOCK_T,),
        in_specs=[
            pl.BlockSpec((BLOCK_T, HIDDEN), lambda i: (i, 0)),
            pl.BlockSpec((NUM_EXPERTS, HIDDEN), lambda i: (0, 0)),
            pl.BlockSpec((1, NUM_EXPERTS), lambda i: (0, 0)),
        ],
        out_specs=[
            pl.BlockSpec((BLOCK_T, TOP_K), lambda i: (i, 0)),
            pl.BlockSpec((BLOCK_T, TOP_K), lambda i: (i, 0)),
        ],
        out_shape=[
            jax.ShapeDtypeStruct((TOKENS, TOP_K), jnp.float32),
            jax.ShapeDtypeStruct((TOKENS, TOP_K), jnp.int32),
        ],
    )(x, W, b.reshape(1, NUM_EXPERTS))


def _sc_scatter_body(p_hbm, i_hbm, scores_hbm, p_v, i_v, scores_v):
    wid = lax.axis_index("s") * NC + lax.axis_index("c")
    base = wid * CHUNK
    pltpu.sync_copy(p_hbm.at[pl.ds(base, CHUNK)], p_v)
    pltpu.sync_copy(i_hbm.at[pl.ds(base, CHUNK)], i_v)

    iota = lax.broadcasted_iota(jnp.int32, (LANES,), 0)
    zf = jnp.zeros((LANES,), jnp.float32)
    zi = jnp.zeros((LANES,), jnp.int32)

    for g in range(GROUPS):
        t0 = g * LANES
        for j in range(LANES):
            for k in range(NUM_EXPERTS // LANES):
                scores_v[t0 + j, pl.ds(k * LANES, LANES)] = zf
        tok = t0 + iota
        p1 = plsc.load_gather(p_v, [tok, zi])
        p2 = plsc.load_gather(p_v, [tok, zi + 1])
        i1 = plsc.load_gather(i_v, [tok, zi])
        i2 = plsc.load_gather(i_v, [tok, zi + 1])
        plsc.store_scatter(scores_v, [tok, i1], p1)
        plsc.store_scatter(scores_v, [tok, i2], p2)

    pltpu.sync_copy(scores_v, scores_hbm.at[pl.ds(base, CHUNK)])


_sc_scatter = functools.partial(
    pl.kernel,
    out_type=jax.ShapeDtypeStruct((TOKENS, NUM_EXPERTS), jnp.float32),
    mesh=plsc.VectorSubcoreMesh(core_axis_name="c", subcore_axis_name="s",
                                num_cores=NC, num_subcores=NS),
    scratch_types=[
        pltpu.VMEM((CHUNK, TOP_K), jnp.float32),
        pltpu.VMEM((CHUNK, TOP_K), jnp.int32),
        pltpu.VMEM((CHUNK, NUM_EXPERTS), jnp.float32),
    ],
    compiler_params=pltpu.CompilerParams(needs_layout_passes=False),
)(_sc_scatter_body)


def kernel(hidden_states, W, b):
    x = hidden_states.reshape(-1, HIDDEN)
    pv, iv = _tc_topk(x, W, b)
    scores = _sc_scatter(pv, iv)
    return scores, iv


# speedup vs baseline: 1.2873x; 1.1083x over previous
import functools

import jax
import jax.numpy as jnp
from jax import lax
from jax.experimental import pallas as pl
from jax.experimental.pallas import tpu as pltpu
from jax.experimental.pallas import tpu_sc as plsc

TOP_K = 2
NUM_EXPERTS = 64
HIDDEN = 2048
TOKENS = 8192
BLOCK_T = 1024

NC = 2
NS = 16
LANES = 16
NW = NC * NS
CHUNK = TOKENS // NW
GROUPS = CHUNK // LANES


def _tc_topk_block(x_ref, w_ref, b_ref, p_ref, i_ref):
    logits = lax.dot_general(
        x_ref[...], w_ref[...], (((1,), (1,)), ((), ())),
        preferred_element_type=jnp.float32)
    logits = logits + b_ref[...]

    eidx = lax.broadcasted_iota(jnp.int32, logits.shape, 1)
    m1 = jnp.max(logits, axis=1, keepdims=True)
    i1 = jnp.min(jnp.where(logits == m1, eidx, NUM_EXPERTS), axis=1,
                 keepdims=True)
    masked = jnp.where(eidx == i1, -jnp.inf, logits)
    m2 = jnp.max(masked, axis=1, keepdims=True)
    i2 = jnp.min(jnp.where(masked == m2, eidx, NUM_EXPERTS), axis=1,
                 keepdims=True)

    d = jnp.exp(m2 - m1)
    p1 = 1.0 / (1.0 + d)
    p2 = d * p1

    p_ref[...] = jnp.concatenate([p1, p2], axis=1)
    i_ref[...] = jnp.concatenate([i1, i2], axis=1)


def _tc_topk(x, W, b):
    return pl.pallas_call(
        _tc_topk_block,
        grid=(TOKENS // BLOCK_T,),
        in_specs=[
            pl.BlockSpec((BLOCK_T, HIDDEN), lambda i: (i, 0)),
            pl.BlockSpec((NUM_EXPERTS, HIDDEN), lambda i: (0, 0)),
            pl.BlockSpec((1, NUM_EXPERTS), lambda i: (0, 0)),
        ],
        out_specs=[
            pl.BlockSpec((BLOCK_T, TOP_K), lambda i: (i, 0)),
            pl.BlockSpec((BLOCK_T, TOP_K), lambda i: (i, 0)),
        ],
        out_shape=[
            jax.ShapeDtypeStruct((TOKENS, TOP_K), jnp.float32),
            jax.ShapeDtypeStruct((TOKENS, TOP_K), jnp.int32),
        ],
    )(x, W, b.reshape(1, NUM_EXPERTS))


def _sc_scatter_body(p_hbm, i_hbm, scores_hbm, p_v, i_v, scores_v):
    wid = lax.axis_index("s") * NC + lax.axis_index("c")
    base = wid * CHUNK
    pltpu.sync_copy(p_hbm.at[pl.ds(base, CHUNK)], p_v)
    pltpu.sync_copy(i_hbm.at[pl.ds(base, CHUNK)], i_v)

    iota = lax.broadcasted_iota(jnp.int32, (LANES,), 0)
    zf = jnp.zeros((LANES,), jnp.float32)
    zi = jnp.zeros((LANES,), jnp.int32)

    def group_body(g, carry):
        t0 = g * LANES
        for j in range(LANES):
            for k in range(NUM_EXPERTS // LANES):
                scores_v[t0 + j, pl.ds(k * LANES, LANES)] = zf
        tok = t0 + iota
        p1 = plsc.load_gather(p_v, [tok, zi])
        p2 = plsc.load_gather(p_v, [tok, zi + 1])
        i1 = plsc.load_gather(i_v, [tok, zi])
        i2 = plsc.load_gather(i_v, [tok, zi + 1])
        plsc.store_scatter(scores_v, [tok, i1], p1)
        plsc.store_scatter(scores_v, [tok, i2], p2)
        return carry

    lax.fori_loop(0, GROUPS, group_body, 0)

    pltpu.sync_copy(scores_v, scores_hbm.at[pl.ds(base, CHUNK)])


_sc_scatter = functools.partial(
    pl.kernel,
    out_type=jax.ShapeDtypeStruct((TOKENS, NUM_EXPERTS), jnp.float32),
    mesh=plsc.VectorSubcoreMesh(core_axis_name="c", subcore_axis_name="s",
                                num_cores=NC, num_subcores=NS),
    scratch_types=[
        pltpu.VMEM((CHUNK, TOP_K), jnp.float32),
        pltpu.VMEM((CHUNK, TOP_K), jnp.int32),
        pltpu.VMEM((CHUNK, NUM_EXPERTS), jnp.float32),
    ],
    compiler_params=pltpu.CompilerParams(needs_layout_passes=False),
)(_sc_scatter_body)


def kernel(hidden_states, W, b):
    x = hidden_states.reshape(-1, HIDDEN)
    pv, iv = _tc_topk(x, W, b)
    scores = _sc_scatter(pv, iv)
    return scores, iv
